# Initial kernel scaffold; baseline (speedup 1.0000x reference)
import jax
import jax.numpy as jnp
from jax.experimental import pallas as pl


def kernel(returns):
    raise NotImplementedError("write your pallas kernel here")


# speedup vs baseline: 21.0489x; 21.0489x over previous
import functools

import jax
import jax.numpy as jnp
from jax import lax
from jax.experimental import pallas as pl
from jax.experimental.pallas import tpu as pltpu

_ALPHA = 0.05
_TARGET = -0.01
_HUBER_DELTA = 0.01

_N = 1048576
_K = max(1, int(_N * _ALPHA))
_ROWS = _N // 128
_MININT = -(2**31)


def _huber(x):
    a = jnp.abs(x)
    return jnp.where(a <= _HUBER_DELTA, 0.5 * x * x,
                     _HUBER_DELTA * (a - 0.5 * _HUBER_DELTA))


def _select_kernel(x_ref, cvar_ref, viol_ref, key_ref):
    x = x_ref[...]
    b = lax.bitcast_convert_type(x, jnp.int32)
    key = b ^ ((b >> 31) & jnp.int32(0x7FFFFFFF))
    key_ref[...] = key

    kth = jnp.int32(_K)

    def bit_step(j, prefix):
        bit = jnp.left_shift(jnp.int32(1), 31 - j)
        cand_u = prefix | bit
        cand_s = cand_u ^ jnp.int32(_MININT)
        cnt = jnp.sum((key_ref[...] < cand_s).astype(jnp.int32))
        return jnp.where(cnt >= kth, prefix, cand_u)

    prefix = lax.fori_loop(0, 32, bit_step, jnp.int32(0), unroll=False)
    t_key = prefix ^ jnp.int32(_MININT)
    t_bits = t_key ^ ((t_key >> 31) & jnp.int32(0x7FFFFFFF))
    t_val = lax.bitcast_convert_type(t_bits, jnp.float32)

    keys = key_ref[...]
    below = keys < t_key
    s_below = jnp.sum(jnp.where(below, _huber(x), jnp.float32(0.0)))
    c_below = jnp.sum(below.astype(jnp.int32))

    total = s_below + (kth - c_below).astype(jnp.float32) * _huber(t_val)
    cvar = -total / jnp.float32(_K)
    viol = jnp.maximum(jnp.float32(_TARGET) - cvar, 0.0) * jnp.float32(5.0)
    cvar_ref[...] = jnp.full((8, 128), cvar, dtype=jnp.float32)
    viol_ref[...] = jnp.full((8, 128), viol, dtype=jnp.float32)


@jax.jit
def kernel(returns):
    x2d = returns.reshape(_ROWS, 128)
    cvar, viol = pl.pallas_call(
        _select_kernel,
        out_shape=[
            jax.ShapeDtypeStruct((8, 128), jnp.float32),
            jax.ShapeDtypeStruct((8, 128), jnp.float32),
        ],
        scratch_shapes=[pltpu.VMEM((_ROWS, 128), jnp.int32)],
    )(x2d)
    return (cvar[0, 0], viol[0, 0])
